# baseline (speedup 1.0000x reference)
import functools

import jax
import jax.numpy as jnp
from jax import lax
from jax.experimental import pallas as pl
from jax.experimental.pallas import tpu as pltpu

_WS = 7
_K = 9


def _sigmoid(x):
    return 1.0 / (1.0 + jnp.exp(-x))


def _graph_body(fT, consts, *, n):
    f2 = jnp.transpose(fT, (1, 0))
    nrm = jnp.sqrt(jnp.sum(f2 * f2, axis=0, keepdims=True))
    nrm = jnp.maximum(nrm, 1e-8)
    xn = f2 / nrm
    sim = lax.dot_general(xn, xn, (((0,), (0,)), ((), ())),
                          preferred_element_type=jnp.float32)

    (alpha22, sdist_bias, mlp, sigma) = consts
    w1m, w1d, b1, w2, b2 = mlp

    ki = lax.convert_element_type(jnp.round(sim * alpha22 + sdist_bias),
                                  jnp.int32)
    rows = lax.broadcasted_iota(jnp.int32, (n, n), 0)
    keyf = lax.bitcast_convert_type(
        (ki << 8) + (jnp.int32(255) - rows), jnp.float32)

    wd = jnp.zeros((n, n), jnp.float32)
    wsum = jnp.zeros((1, n), jnp.float32)

    for _ in range(_K):
        kmaxf = jnp.max(keyf, axis=0, keepdims=True)
        first = keyf == kmaxf
        keyf = jnp.where(first, jnp.float32(0.0), keyf)
        kmax = lax.bitcast_convert_type(kmaxf, jnp.int32)
        jsel = jnp.int32(255) - (kmax & jnp.int32(255))
        m = (lax.convert_element_type(kmax >> 8, jnp.float32)
             * jnp.float32(1.0 / (3.0 * 2.0**20)) - jnp.float32(1.0))
        i_row = lax.broadcasted_iota(jnp.int32, (1, n), 1)
        dr = (i_row >> 4) - (jsel >> 4)
        dc = (i_row & 15) - (jsel & 15)
        e2 = (dr * dr + dc * dc).astype(jnp.float32)
        sd_t = jnp.exp(-e2 / (2.0 * sigma * sigma))
        u = b2
        for c_ in range(4):
            h = m * w1m[c_] + sd_t * w1d[c_] + b1[c_]
            h = h * _sigmoid(h)
            u = u + h * w2[c_]
        w_t = _sigmoid(u)
        wd = jnp.where(first, w_t, wd)
        wsum = wsum + w_t

    wdn = wd * (1.0 / (wsum + 1e-12))
    return lax.dot_general(wdn, fT, (((0,), (0,)), ((), ())),
                           preferred_element_type=jnp.float32)


def _strip_body(scal_ref, x_ref, fwt_ref, fb_ref, out_ref, *, n, hp, wp, ws):
    d4 = fwt_ref.shape[1]
    fT = lax.dot_general(x_ref[...], fwt_ref[...], (((0,), (0,)), ((), ())),
                         preferred_element_type=jnp.float32) + fb_ref[...]
    f4 = fT.reshape(hp, ws, wp, d4)

    sigma = scal_ref[0]
    alpha = scal_ref[1]
    w1m = [scal_ref[2 + 2 * c_] / alpha for c_ in range(4)]
    w1d = [scal_ref[3 + 2 * c_]
           - scal_ref[2 + 2 * c_] * (1.0 - alpha) / alpha for c_ in range(4)]
    b1 = [scal_ref[10 + c_] for c_ in range(4)]
    w2 = [scal_ref[14 + c_] for c_ in range(4)]
    b2 = scal_ref[18]

    rows = lax.broadcasted_iota(jnp.int32, (n, n), 0)
    cols = lax.broadcasted_iota(jnp.int32, (n, n), 1)
    dr = (rows >> 4) - (cols >> 4)
    dc = (rows & 15) - (cols & 15)
    d2 = (dr * dr + dc * dc).astype(jnp.float32)
    dist = jnp.sqrt(d2)
    sdist = jnp.exp(-(dist * dist) / (2.0 * sigma * sigma))
    scale = jnp.float32(3.0 * 2.0**20)
    alpha22 = alpha * scale
    sdist_bias = (1.0 - alpha) * scale * sdist + scale

    consts = (alpha22, sdist_bias, (w1m, w1d, b1, w2, b2), sigma)

    pieces = []
    for hg in range(ws):
        fTp = f4[:, hg].reshape(n, d4)
        odTp = _graph_body(fTp, consts, n=n)
        pieces.append(odTp.reshape(hp, 1, wp, d4))
    out_ref[0] = jnp.concatenate(pieces, axis=1)


def _proj_body(od_ref, pw_ref, pb_ref, out_ref):
    odc = jnp.transpose(od_ref[...], (1, 0))
    oc = jnp.dot(pw_ref[...], odc, preferred_element_type=jnp.float32)
    out_ref[...] = oc + pb_ref[...]


def kernel(x_in, sigma, alpha, f_w, f_b, p_w, p_b, mlp_w1, mlp_b1, mlp_w2, mlp_b2):
    B, C, H, W = x_in.shape
    ws = _WS
    hp, wp = H // ws, W // ws
    n = hp * wp
    d4 = f_w.shape[0]
    strip = hp * W

    X = x_in.reshape(C, H * W)
    fw_t = f_w.T
    scal = jnp.concatenate([
        sigma.reshape(1), alpha.reshape(1),
        mlp_w1.reshape(-1), mlp_b1.reshape(-1),
        mlp_w2.reshape(-1), mlp_b2.reshape(-1),
    ]).astype(jnp.float32)

    sbody = functools.partial(_strip_body, n=n, hp=hp, wp=wp, ws=ws)
    od = pl.pallas_call(
        sbody,
        grid=(ws,),
        in_specs=[
            pl.BlockSpec(memory_space=pltpu.SMEM),
            pl.BlockSpec((C, strip), lambda i: (0, i)),
            pl.BlockSpec((C, d4), lambda i: (0, 0)),
            pl.BlockSpec((1, d4), lambda i: (0, 0)),
        ],
        out_specs=pl.BlockSpec((1, hp, ws, wp, d4), lambda i: (i, 0, 0, 0, 0)),
        out_shape=jax.ShapeDtypeStruct((ws, hp, ws, wp, d4), jnp.float32),
    )(scal, X, fw_t, f_b.reshape(1, d4))

    od2 = od.reshape(H * W, d4)
    y = pl.pallas_call(
        _proj_body,
        grid=(ws,),
        in_specs=[
            pl.BlockSpec((strip, d4), lambda i: (i, 0)),
            pl.BlockSpec((C, d4), lambda i: (0, 0)),
            pl.BlockSpec((C, 1), lambda i: (0, 0)),
        ],
        out_specs=pl.BlockSpec((C, strip), lambda i: (0, i)),
        out_shape=jax.ShapeDtypeStruct((C, H * W), jnp.float32),
    )(od2, p_w, p_b.reshape(C, 1))

    return y.reshape(B, C, H * W)


# speedup vs baseline: 1.0118x; 1.0118x over previous
import functools

import jax
import jax.numpy as jnp
from jax import lax
from jax.experimental import pallas as pl
from jax.experimental.pallas import tpu as pltpu

_WS = 7
_K = 9


def _sigmoid(x):
    return 1.0 / (1.0 + jnp.exp(-x))


def _mlp_weight(m, sd_t, mlp):
    w1m, w1d, b1, w2, b2 = mlp
    u = b2
    for c_ in range(4):
        h = m * w1m[c_] + sd_t * w1d[c_] + b1[c_]
        h = h * _sigmoid(h)
        u = u + h * w2[c_]
    return _sigmoid(u)


def _graph_body(fT, consts, *, n):
    f2 = jnp.transpose(fT, (1, 0))
    s2 = jnp.sum(f2 * f2, axis=0, keepdims=True)
    nrm0 = jnp.sqrt(s2)
    nrm = jnp.maximum(nrm0, 1e-8)
    xn = f2 / nrm
    sim = lax.dot_general(xn, xn, (((0,), (0,)), ((), ())),
                          preferred_element_type=jnp.float32)

    (alpha22, sdist_bias, diag, mlp, sigma, alpha) = consts

    ki = lax.convert_element_type(jnp.round(sim * alpha22 + sdist_bias),
                                  jnp.int32)
    rows = lax.broadcasted_iota(jnp.int32, (n, n), 0)
    keyf = lax.bitcast_convert_type(
        (ki << 8) + (jnp.int32(255) - rows), jnp.float32)

    ratio = nrm0 / nrm
    m0 = alpha * (ratio * ratio) + (1.0 - alpha)
    w_t0 = _mlp_weight(m0, jnp.float32(1.0), mlp)
    wd = jnp.where(diag, w_t0, jnp.float32(0.0))
    wsum = w_t0

    for _ in range(_K - 1):
        kmaxf = jnp.max(keyf, axis=0, keepdims=True)
        first = keyf == kmaxf
        keyf = jnp.where(first, jnp.float32(0.0), keyf)
        kmax = lax.bitcast_convert_type(kmaxf, jnp.int32)
        jsel = jnp.int32(255) - (kmax & jnp.int32(255))
        m = (lax.convert_element_type(kmax >> 8, jnp.float32)
             * jnp.float32(1.0 / (3.0 * 2.0**20)) - jnp.float32(1.0))
        i_row = lax.broadcasted_iota(jnp.int32, (1, n), 1)
        dr = (i_row >> 4) - (jsel >> 4)
        dc = (i_row & 15) - (jsel & 15)
        e2 = (dr * dr + dc * dc).astype(jnp.float32)
        sd_t = jnp.exp(-e2 / (2.0 * sigma * sigma))
        w_t = _mlp_weight(m, sd_t, mlp)
        wd = jnp.where(first, w_t, wd)
        wsum = wsum + w_t

    wdn = wd * (1.0 / (wsum + 1e-12))
    return lax.dot_general(wdn, fT, (((0,), (0,)), ((), ())),
                           preferred_element_type=jnp.float32)


def _strip_body(scal_ref, x_ref, fwt_ref, fb_ref, out_ref, *, n, hp, wp, ws):
    d4 = fwt_ref.shape[1]
    fT = lax.dot_general(x_ref[...], fwt_ref[...], (((0,), (0,)), ((), ())),
                         preferred_element_type=jnp.float32) + fb_ref[...]
    f4 = fT.reshape(hp, ws, wp, d4)

    sigma = scal_ref[0]
    alpha = scal_ref[1]
    w1m = [scal_ref[2 + 2 * c_] / alpha for c_ in range(4)]
    w1d = [scal_ref[3 + 2 * c_]
           - scal_ref[2 + 2 * c_] * (1.0 - alpha) / alpha for c_ in range(4)]
    b1 = [scal_ref[10 + c_] for c_ in range(4)]
    w2 = [scal_ref[14 + c_] for c_ in range(4)]
    b2 = scal_ref[18]

    rows = lax.broadcasted_iota(jnp.int32, (n, n), 0)
    cols = lax.broadcasted_iota(jnp.int32, (n, n), 1)
    dr = (rows >> 4) - (cols >> 4)
    dc = (rows & 15) - (cols & 15)
    d2 = (dr * dr + dc * dc).astype(jnp.float32)
    dist = jnp.sqrt(d2)
    sdist = jnp.exp(-(dist * dist) / (2.0 * sigma * sigma))
    scale = jnp.float32(3.0 * 2.0**20)
    alpha22 = alpha * scale
    diag = rows == cols
    sdist_bias = ((1.0 - alpha) * scale * sdist + scale
                  - jnp.where(diag, jnp.float32(7.0 * 2.0**20), jnp.float32(0.0)))

    consts = (alpha22, sdist_bias, diag, (w1m, w1d, b1, w2, b2), sigma, alpha)

    pieces = []
    for hg in range(ws):
        fTp = f4[:, hg].reshape(n, d4)
        odTp = _graph_body(fTp, consts, n=n)
        pieces.append(odTp.reshape(hp, 1, wp, d4))
    out_ref[0] = jnp.concatenate(pieces, axis=1)


def _proj_body(od_ref, pw_ref, pb_ref, out_ref):
    odc = jnp.transpose(od_ref[...], (1, 0))
    oc = jnp.dot(pw_ref[...], odc, preferred_element_type=jnp.float32)
    out_ref[...] = oc + pb_ref[...]


def kernel(x_in, sigma, alpha, f_w, f_b, p_w, p_b, mlp_w1, mlp_b1, mlp_w2, mlp_b2):
    B, C, H, W = x_in.shape
    ws = _WS
    hp, wp = H // ws, W // ws
    n = hp * wp
    d4 = f_w.shape[0]
    strip = hp * W

    X = x_in.reshape(C, H * W)
    fw_t = f_w.T
    scal = jnp.concatenate([
        sigma.reshape(1), alpha.reshape(1),
        mlp_w1.reshape(-1), mlp_b1.reshape(-1),
        mlp_w2.reshape(-1), mlp_b2.reshape(-1),
    ]).astype(jnp.float32)

    sbody = functools.partial(_strip_body, n=n, hp=hp, wp=wp, ws=ws)
    od = pl.pallas_call(
        sbody,
        grid=(ws,),
        in_specs=[
            pl.BlockSpec(memory_space=pltpu.SMEM),
            pl.BlockSpec((C, strip), lambda i: (0, i)),
            pl.BlockSpec((C, d4), lambda i: (0, 0)),
            pl.BlockSpec((1, d4), lambda i: (0, 0)),
        ],
        out_specs=pl.BlockSpec((1, hp, ws, wp, d4), lambda i: (i, 0, 0, 0, 0)),
        out_shape=jax.ShapeDtypeStruct((ws, hp, ws, wp, d4), jnp.float32),
    )(scal, X, fw_t, f_b.reshape(1, d4))

    od2 = od.reshape(H * W, d4)
    y = pl.pallas_call(
        _proj_body,
        grid=(ws,),
        in_specs=[
            pl.BlockSpec((strip, d4), lambda i: (i, 0)),
            pl.BlockSpec((C, d4), lambda i: (0, 0)),
            pl.BlockSpec((C, 1), lambda i: (0, 0)),
        ],
        out_specs=pl.BlockSpec((C, strip), lambda i: (0, i)),
        out_shape=jax.ShapeDtypeStruct((C, H * W), jnp.float32),
    )(od2, p_w, p_b.reshape(C, 1))

    return y.reshape(B, C, H * W)
